# initial kernel scaffold (unmeasured)
import jax
import jax.numpy as jnp
from jax import lax
from jax.experimental import pallas as pl
from jax.experimental.pallas import tpu as pltpu

N_DEV = 16


def kernel(x, w_mat):
    m, _ = x.shape
    _, n = w_mat.shape
    chunk = m // N_DEV
    n_hops = 2 * (N_DEV - 1)

    def body(x_ref, w_ref, out_ref, send_buf, recv_buf, send_sems, recv_sems,
             credit_sem):
        my = lax.axis_index("i")
        left = (my + N_DEV - 1) % N_DEV
        right = (my + 1) % N_DEV

        out_ref[:, :] = jnp.dot(
            x_ref[:, :].astype(jnp.bfloat16),
            w_ref[:, :].astype(jnp.bfloat16),
            preferred_element_type=jnp.float32,
        )

        send_buf[0, :, :] = out_ref[pl.ds(my * chunk, chunk), :].astype(
            jnp.bfloat16)

        barrier_sem = pltpu.get_barrier_semaphore()
        for nbr in (left, right):
            pl.semaphore_signal(
                barrier_sem, inc=1,
                device_id=(nbr,), device_id_type=pl.DeviceIdType.MESH,
            )
        pl.semaphore_wait(barrier_sem, 2)

        for h in range(n_hops):
            slot = h % 2
            nxt = (h + 1) % 2
            if h >= 2:
                pl.semaphore_wait(credit_sem, 1)
            rdma = pltpu.make_async_remote_copy(
                src_ref=send_buf.at[slot],
                dst_ref=recv_buf.at[slot],
                send_sem=send_sems.at[slot],
                recv_sem=recv_sems.at[slot],
                device_id=(right,),
                device_id_type=pl.DeviceIdType.MESH,
            )
            rdma.start()
            rdma.wait()

            if h < N_DEV - 1:
                c = (my - h - 1) % N_DEV
                acc = (recv_buf[slot, :, :].astype(jnp.float32)
                       + out_ref[pl.ds(c * chunk, chunk), :])
                if h == N_DEV - 2:
                    out_ref[pl.ds(c * chunk, chunk), :] = acc
                send_buf[nxt, :, :] = acc.astype(jnp.bfloat16)
            else:
                a = h - (N_DEV - 1)
                c = (my - a) % N_DEV
                out_ref[pl.ds(c * chunk, chunk), :] = (
                    recv_buf[slot, :, :].astype(jnp.float32))
                if h < n_hops - 1:
                    send_buf[nxt, :, :] = recv_buf[slot, :, :]

            if h < n_hops - 2:
                pl.semaphore_signal(
                    credit_sem, inc=1,
                    device_id=(left,), device_id_type=pl.DeviceIdType.MESH,
                )

        y = out_ref[:, :]
        amax = jnp.max(jnp.abs(y))
        scale = amax / 127.0
        q = jnp.clip(jnp.round(y / scale), -127.0, 127.0)
        out_ref[:, :] = q * scale

    return pl.pallas_call(
        body,
        out_shape=jax.ShapeDtypeStruct((m, n), jnp.float32),
        in_specs=[
            pl.BlockSpec(memory_space=pltpu.VMEM),
            pl.BlockSpec(memory_space=pltpu.VMEM),
        ],
        out_specs=pl.BlockSpec(memory_space=pltpu.VMEM),
        scratch_shapes=[
            pltpu.VMEM((2, chunk, n), jnp.bfloat16),
            pltpu.VMEM((2, chunk, n), jnp.bfloat16),
            pltpu.SemaphoreType.DMA((2,)),
            pltpu.SemaphoreType.DMA((2,)),
            pltpu.SemaphoreType.REGULAR,
        ],
        compiler_params=pltpu.CompilerParams(collective_id=0),
    )(x, w_mat)


# baseline (device time: 301757 ns/iter reference)
import jax
import jax.numpy as jnp
from jax import lax
from jax.experimental import pallas as pl
from jax.experimental.pallas import tpu as pltpu

N_DEV = 16


def kernel(x, w_mat):
    m, _ = x.shape
    _, n = w_mat.shape
    chunk = m // N_DEV
    half = n // 2
    n_hops = N_DEV - 1
    CL = pl.ds(0, half)
    CH = pl.ds(half, half)

    def body(x_ref, w_ref, out_ref,
             fsend, frecv, rsend, rrecv,
             fsend_sems, frecv_sems, rsend_sems, rrecv_sems,
             fcredit, rcredit,
             bcast, bsend_sems, brecv_sems,
             fq_own, rq_own, fag_recv, rag_recv,
             fag_send_sems, fag_recv_sems, rag_send_sems, rag_recv_sems,
             fag_credit, rag_credit):
        my = lax.axis_index("i")
        left = (my + N_DEV - 1) % N_DEV
        right = (my + 1) % N_DEV

        def rows(c):
            return pl.ds(c * chunk, chunk)

        wb = w_ref[:, :].astype(jnp.bfloat16)
        for j in range(N_DEV):
            out_ref[rows(j), :] = jnp.dot(
                x_ref[pl.ds(j * chunk, chunk), :].astype(jnp.bfloat16),
                wb,
                preferred_element_type=jnp.float32,
            )

        fsend[0, :, :] = out_ref[rows(my), CL].astype(jnp.bfloat16)
        rsend[0, :, :] = out_ref[rows(my), CH].astype(jnp.bfloat16)

        barrier_sem = pltpu.get_barrier_semaphore()
        for nbr in (left, right):
            pl.semaphore_signal(
                barrier_sem, inc=1,
                device_id=(nbr,), device_id_type=pl.DeviceIdType.MESH,
            )
        pl.semaphore_wait(barrier_sem, 2)

        for h in range(n_hops):
            slot = h % 2
            nxt = (h + 1) % 2
            if h >= 2:
                pl.semaphore_wait(fcredit, 1)
                pl.semaphore_wait(rcredit, 1)
            f_rdma = pltpu.make_async_remote_copy(
                src_ref=fsend.at[slot], dst_ref=frecv.at[slot],
                send_sem=fsend_sems.at[slot], recv_sem=frecv_sems.at[slot],
                device_id=(right,), device_id_type=pl.DeviceIdType.MESH,
            )
            r_rdma = pltpu.make_async_remote_copy(
                src_ref=rsend.at[slot], dst_ref=rrecv.at[slot],
                send_sem=rsend_sems.at[slot], recv_sem=rrecv_sems.at[slot],
                device_id=(left,), device_id_type=pl.DeviceIdType.MESH,
            )
            f_rdma.start()
            r_rdma.start()
            f_rdma.wait()
            r_rdma.wait()

            cf = (my - h - 1) % N_DEV
            cr = (my + h + 1) % N_DEV
            acc_f = (frecv[slot, :, :].astype(jnp.float32)
                     + out_ref[rows(cf), CL])
            acc_r = (rrecv[slot, :, :].astype(jnp.float32)
                     + out_ref[rows(cr), CH])
            if h < n_hops - 1:
                fsend[nxt, :, :] = acc_f.astype(jnp.bfloat16)
                rsend[nxt, :, :] = acc_r.astype(jnp.bfloat16)
            else:
                out_ref[rows(cf), CL] = acc_f
                out_ref[rows(cr), CH] = acc_r
            if h <= n_hops - 3:
                pl.semaphore_signal(
                    fcredit, inc=1,
                    device_id=(left,), device_id_type=pl.DeviceIdType.MESH,
                )
                pl.semaphore_signal(
                    rcredit, inc=1,
                    device_id=(right,), device_id_type=pl.DeviceIdType.MESH,
                )

        own_f = (my + 1) % N_DEV
        own_r = (my + N_DEV - 1) % N_DEV

        amax_own = jnp.maximum(
            jnp.max(jnp.abs(out_ref[rows(own_f), CL])),
            jnp.max(jnp.abs(out_ref[rows(own_r), CH])),
        )
        bcast[0, :, :] = jnp.full((8, 128), amax_own, jnp.float32)
        bsends = []
        for k in range(1, N_DEV):
            s = N_DEV - k
            b = pltpu.make_async_remote_copy(
                src_ref=bcast.at[0], dst_ref=bcast.at[s],
                send_sem=bsend_sems.at[s], recv_sem=brecv_sems.at[s],
                device_id=((my + k) % N_DEV,),
                device_id_type=pl.DeviceIdType.MESH,
            )
            b.start()
            bsends.append(b)
        for s in range(1, N_DEV):
            pltpu.make_async_remote_copy(
                src_ref=bcast.at[0], dst_ref=bcast.at[s],
                send_sem=bsend_sems.at[0], recv_sem=brecv_sems.at[s],
                device_id=(left,), device_id_type=pl.DeviceIdType.MESH,
            ).wait_recv()
        for b in bsends:
            b.wait_send()

        amax_g = jnp.max(bcast[:, :, :])
        scale = amax_g / 127.0

        yf = out_ref[rows(own_f), CL]
        qf = jnp.clip(jnp.round(yf / scale), -127.0, 127.0)
        fq_own[:, :] = qf.astype(jnp.int8)
        out_ref[rows(own_f), CL] = qf * scale
        yr = out_ref[rows(own_r), CH]
        qr = jnp.clip(jnp.round(yr / scale), -127.0, 127.0)
        rq_own[:, :] = qr.astype(jnp.int8)
        out_ref[rows(own_r), CH] = qr * scale

        for h in range(n_hops):
            slot = h % 2
            if h >= 2:
                pl.semaphore_wait(fag_credit, 1)
                pl.semaphore_wait(rag_credit, 1)
            f_src = fq_own if h == 0 else fag_recv.at[(h - 1) % 2]
            r_src = rq_own if h == 0 else rag_recv.at[(h - 1) % 2]
            f_ag = pltpu.make_async_remote_copy(
                src_ref=f_src, dst_ref=fag_recv.at[slot],
                send_sem=fag_send_sems.at[slot],
                recv_sem=fag_recv_sems.at[slot],
                device_id=(right,), device_id_type=pl.DeviceIdType.MESH,
            )
            r_ag = pltpu.make_async_remote_copy(
                src_ref=r_src, dst_ref=rag_recv.at[slot],
                send_sem=rag_send_sems.at[slot],
                recv_sem=rag_recv_sems.at[slot],
                device_id=(left,), device_id_type=pl.DeviceIdType.MESH,
            )
            f_ag.start()
            r_ag.start()
            f_ag.wait()
            r_ag.wait()
            if 1 <= h <= n_hops - 2:
                pl.semaphore_signal(
                    fag_credit, inc=1,
                    device_id=(left,), device_id_type=pl.DeviceIdType.MESH,
                )
                pl.semaphore_signal(
                    rag_credit, inc=1,
                    device_id=(right,), device_id_type=pl.DeviceIdType.MESH,
                )
            out_ref[rows((my - h) % N_DEV), CL] = (
                fag_recv[slot, :, :].astype(jnp.float32) * scale)
            out_ref[rows((my + h) % N_DEV), CH] = (
                rag_recv[slot, :, :].astype(jnp.float32) * scale)

    return pl.pallas_call(
        body,
        out_shape=jax.ShapeDtypeStruct((m, n), jnp.float32),
        in_specs=[
            pl.BlockSpec(memory_space=pltpu.VMEM),
            pl.BlockSpec(memory_space=pltpu.VMEM),
        ],
        out_specs=pl.BlockSpec(memory_space=pltpu.VMEM),
        scratch_shapes=[
            pltpu.VMEM((2, chunk, half), jnp.bfloat16),
            pltpu.VMEM((2, chunk, half), jnp.bfloat16),
            pltpu.VMEM((2, chunk, half), jnp.bfloat16),
            pltpu.VMEM((2, chunk, half), jnp.bfloat16),
            pltpu.SemaphoreType.DMA((2,)),
            pltpu.SemaphoreType.DMA((2,)),
            pltpu.SemaphoreType.DMA((2,)),
            pltpu.SemaphoreType.DMA((2,)),
            pltpu.SemaphoreType.REGULAR,
            pltpu.SemaphoreType.REGULAR,
            pltpu.VMEM((N_DEV, 8, 128), jnp.float32),
            pltpu.SemaphoreType.DMA((N_DEV,)),
            pltpu.SemaphoreType.DMA((N_DEV,)),
            pltpu.VMEM((chunk, half), jnp.int8),
            pltpu.VMEM((chunk, half), jnp.int8),
            pltpu.VMEM((2, chunk, half), jnp.int8),
            pltpu.VMEM((2, chunk, half), jnp.int8),
            pltpu.SemaphoreType.DMA((2,)),
            pltpu.SemaphoreType.DMA((2,)),
            pltpu.SemaphoreType.DMA((2,)),
            pltpu.SemaphoreType.DMA((2,)),
            pltpu.SemaphoreType.REGULAR,
            pltpu.SemaphoreType.REGULAR,
        ],
        compiler_params=pltpu.CompilerParams(
            collective_id=0,
            vmem_limit_bytes=60 * 1024 * 1024,
        ),
    )(x, w_mat)


# device time: 256747 ns/iter; 1.1753x vs baseline; 1.1753x over previous
import jax
import jax.numpy as jnp
from jax import lax
from jax.experimental import pallas as pl
from jax.experimental.pallas import tpu as pltpu

N_DEV = 16
N_CH = 4


def kernel(x, w_mat):
    m, _ = x.shape
    _, n = w_mat.shape
    chunk = m // N_DEV
    half = n // 2
    quar = n // 4
    n_hops = N_DEV - 1

    def body(x_ref, w_ref, out_ref, *scr):
        rs = [dict(send=scr[5 * i], recv=scr[5 * i + 1], ssem=scr[5 * i + 2],
                   rsem=scr[5 * i + 3], credit=scr[5 * i + 4])
              for i in range(N_CH)]
        bcast, bsend_sems, brecv_sems = scr[20], scr[21], scr[22]
        ag = [dict(own=scr[23 + 5 * i], recv=scr[24 + 5 * i],
                   ssem=scr[25 + 5 * i], rsem=scr[26 + 5 * i],
                   credit=scr[27 + 5 * i])
              for i in range(N_CH)]

        my = lax.axis_index("i")
        left = (my + N_DEV - 1) % N_DEV
        right = (my + 1) % N_DEV

        def rows(c):
            return pl.ds(c * chunk, chunk)

        for ci in range(N_CH):
            fwd = ci < 2
            cols = pl.ds(quar * ci, quar)
            peer = right if fwd else left
            up = left if fwd else right
            for d in (rs[ci], ag[ci]):
                d["cols"] = cols
                d["peer"] = peer
                d["up"] = up
                d["fwd"] = fwd
        order = [0, 2, 1, 3]

        def ch_signal(sem, dev):
            pl.semaphore_signal(
                sem, inc=1, device_id=(dev,),
                device_id_type=pl.DeviceIdType.MESH,
            )

        def rs_rdma(ch, h):
            s = h % 2
            return pltpu.make_async_remote_copy(
                src_ref=ch["send"].at[s], dst_ref=ch["recv"].at[s],
                send_sem=ch["ssem"].at[s], recv_sem=ch["rsem"].at[s],
                device_id=(ch["peer"],),
                device_id_type=pl.DeviceIdType.MESH,
            )

        def ag_rdma(ch, h):
            s = h % 2
            src = ch["own"] if h == 0 else ch["recv"].at[(h - 1) % 2]
            return pltpu.make_async_remote_copy(
                src_ref=src, dst_ref=ch["recv"].at[s],
                send_sem=ch["ssem"].at[s], recv_sem=ch["rsem"].at[s],
                device_id=(ch["peer"],),
                device_id_type=pl.DeviceIdType.MESH,
            )

        wb = w_ref[:, :].astype(jnp.bfloat16)
        wlo = wb[:, 0:half]
        whi = wb[:, half:n]
        out_ref[rows(my), :] = jnp.dot(
            x_ref[pl.ds(my * chunk, chunk), :].astype(jnp.bfloat16),
            wb, preferred_element_type=jnp.float32,
        )
        for ci in order:
            ch = rs[ci]
            ch["send"][0, :, :] = out_ref[rows(my), ch["cols"]].astype(
                jnp.bfloat16)

        barrier_sem = pltpu.get_barrier_semaphore()
        for nbr in (left, right):
            ch_signal(barrier_sem, nbr)
        pl.semaphore_wait(barrier_sem, 2)

        for ci in order:
            r = rs_rdma(rs[ci], 0)
            r.start()
            rs[ci]["rdma"] = [r]

        for h in range(n_hops):
            cf = (my - h - 1) % N_DEV
            cr = (my + h + 1) % N_DEV
            out_ref[rows(cf), pl.ds(0, half)] = jnp.dot(
                x_ref[pl.ds(cf * chunk, chunk), :].astype(jnp.bfloat16),
                wlo, preferred_element_type=jnp.float32,
            )
            out_ref[rows(cr), pl.ds(half, half)] = jnp.dot(
                x_ref[pl.ds(cr * chunk, chunk), :].astype(jnp.bfloat16),
                whi, preferred_element_type=jnp.float32,
            )
            for ci in order:
                ch = rs[ci]
                c = cf if ch["fwd"] else cr
                ch["rdma"][h].wait()
                acc = (ch["recv"][h % 2, :, :].astype(jnp.float32)
                       + out_ref[rows(c), ch["cols"]])
                if h < n_hops - 1:
                    ch["send"][(h + 1) % 2, :, :] = acc.astype(jnp.bfloat16)
                    if h >= 1:
                        pl.semaphore_wait(ch["credit"], 1)
                    nr = rs_rdma(ch, h + 1)
                    nr.start()
                    ch["rdma"].append(nr)
                else:
                    out_ref[rows(c), ch["cols"]] = acc
                if h <= n_hops - 3:
                    ch_signal(ch["credit"], ch["up"])

        own_f = (my + 1) % N_DEV
        own_r = (my + N_DEV - 1) % N_DEV

        amax_own = jnp.maximum(
            jnp.max(jnp.abs(out_ref[rows(own_f), pl.ds(0, half)])),
            jnp.max(jnp.abs(out_ref[rows(own_r), pl.ds(half, half)])),
        )
        bcast[0, :, :] = jnp.full((8, 128), amax_own, jnp.float32)
        bsends = []
        for k in range(1, N_DEV):
            s = N_DEV - k
            b = pltpu.make_async_remote_copy(
                src_ref=bcast.at[0], dst_ref=bcast.at[s],
                send_sem=bsend_sems.at[s], recv_sem=brecv_sems.at[s],
                device_id=((my + k) % N_DEV,),
                device_id_type=pl.DeviceIdType.MESH,
            )
            b.start()
            bsends.append(b)
        for s in range(1, N_DEV):
            pltpu.make_async_remote_copy(
                src_ref=bcast.at[0], dst_ref=bcast.at[s],
                send_sem=bsend_sems.at[0], recv_sem=brecv_sems.at[s],
                device_id=(left,), device_id_type=pl.DeviceIdType.MESH,
            ).wait_recv()
        for b in bsends:
            b.wait_send()

        amax_g = jnp.max(bcast[:, :, :])
        scale = amax_g / 127.0

        yf = out_ref[rows(own_f), pl.ds(0, half)]
        qf = jnp.clip(jnp.round(yf / scale), -127.0, 127.0)
        ag[0]["own"][:, :] = qf[:, 0:quar].astype(jnp.int8)
        ag[1]["own"][:, :] = qf[:, quar:half].astype(jnp.int8)
        out_ref[rows(own_f), pl.ds(0, half)] = qf * scale
        yr = out_ref[rows(own_r), pl.ds(half, half)]
        qr = jnp.clip(jnp.round(yr / scale), -127.0, 127.0)
        ag[2]["own"][:, :] = qr[:, 0:quar].astype(jnp.int8)
        ag[3]["own"][:, :] = qr[:, quar:half].astype(jnp.int8)
        out_ref[rows(own_r), pl.ds(half, half)] = qr * scale

        for ci in order:
            r = ag_rdma(ag[ci], 0)
            r.start()
            ag[ci]["rdma"] = [r]
        for h in range(n_hops):
            for ci in order:
                ch = ag[ci]
                c = (my - h) % N_DEV if ch["fwd"] else (my + h) % N_DEV
                ch["rdma"][h].wait()
                if 1 <= h <= n_hops - 2:
                    ch_signal(ch["credit"], ch["up"])
                if h < n_hops - 1:
                    if h >= 1:
                        pl.semaphore_wait(ch["credit"], 1)
                    nr = ag_rdma(ch, h + 1)
                    nr.start()
                    ch["rdma"].append(nr)
                out_ref[rows(c), ch["cols"]] = (
                    ch["recv"][h % 2, :, :].astype(jnp.float32) * scale)

    rs_scratch = []
    for _ in range(N_CH):
        rs_scratch += [
            pltpu.VMEM((2, chunk, quar), jnp.bfloat16),
            pltpu.VMEM((2, chunk, quar), jnp.bfloat16),
            pltpu.SemaphoreType.DMA((2,)),
            pltpu.SemaphoreType.DMA((2,)),
            pltpu.SemaphoreType.REGULAR,
        ]
    ag_scratch = []
    for _ in range(N_CH):
        ag_scratch += [
            pltpu.VMEM((chunk, quar), jnp.int8),
            pltpu.VMEM((2, chunk, quar), jnp.int8),
            pltpu.SemaphoreType.DMA((2,)),
            pltpu.SemaphoreType.DMA((2,)),
            pltpu.SemaphoreType.REGULAR,
        ]
    return pl.pallas_call(
        body,
        out_shape=jax.ShapeDtypeStruct((m, n), jnp.float32),
        in_specs=[
            pl.BlockSpec(memory_space=pltpu.VMEM),
            pl.BlockSpec(memory_space=pltpu.VMEM),
        ],
        out_specs=pl.BlockSpec(memory_space=pltpu.VMEM),
        scratch_shapes=rs_scratch + [
            pltpu.VMEM((N_DEV, 8, 128), jnp.float32),
            pltpu.SemaphoreType.DMA((N_DEV,)),
            pltpu.SemaphoreType.DMA((N_DEV,)),
        ] + ag_scratch,
        compiler_params=pltpu.CompilerParams(
            collective_id=0,
            vmem_limit_bytes=60 * 1024 * 1024,
        ),
    )(x, w_mat)


# device time: 191768 ns/iter; 1.5736x vs baseline; 1.3388x over previous
import jax
import jax.numpy as jnp
from jax import lax
from jax.experimental import pallas as pl
from jax.experimental.pallas import tpu as pltpu

N_DEV = 16
N_CH = 4
SLOTS = 4

import os
_PROFILE_SCOPES = bool(int(os.environ.get("KERNEL_PROFILE_SCOPES", "0")))
_ABLATE = set(os.environ.get("KERNEL_ABLATE", "").split(","))
_DO_RS = "rs" not in _ABLATE
_DO_AG = "ag" not in _ABLATE
_DO_BC = "bc" not in _ABLATE


def kernel(x, w_mat):
    m, _ = x.shape
    _, n = w_mat.shape
    chunk = m // N_DEV
    half = n // 2
    quar = n // 4
    n_hops = N_DEV - 1

    def body(x_ref, w_ref, out_ref, *scr):
        rs = [dict(send=scr[5 * i], recv=scr[5 * i + 1], ssem=scr[5 * i + 2],
                   rsem=scr[5 * i + 3], credit=scr[5 * i + 4])
              for i in range(N_CH)]
        bcast, bsend_sems, brecv_sems = scr[20], scr[21], scr[22]
        ag = [dict(own=scr[23 + 5 * i], recv=scr[24 + 5 * i],
                   ssem=scr[25 + 5 * i], rsem=scr[26 + 5 * i],
                   credit=scr[27 + 5 * i])
              for i in range(N_CH)]

        my = lax.axis_index("i")
        left = (my + N_DEV - 1) % N_DEV
        right = (my + 1) % N_DEV

        def rows(c):
            return pl.ds(c * chunk, chunk)

        for ci in range(N_CH):
            fwd = ci < 2
            cols = pl.ds(quar * ci, quar)
            peer = right if fwd else left
            up = left if fwd else right
            for d in (rs[ci], ag[ci]):
                d["cols"] = cols
                d["peer"] = peer
                d["up"] = up
                d["fwd"] = fwd
        order = [0, 2, 1, 3]

        def ch_signal(sem, dev):
            pl.semaphore_signal(
                sem, inc=1, device_id=(dev,),
                device_id_type=pl.DeviceIdType.MESH,
            )

        def rs_rdma(ch, h):
            s = h % SLOTS
            return pltpu.make_async_remote_copy(
                src_ref=ch["send"].at[s], dst_ref=ch["recv"].at[s],
                send_sem=ch["ssem"].at[s], recv_sem=ch["rsem"].at[s],
                device_id=(ch["peer"],),
                device_id_type=pl.DeviceIdType.MESH,
            )

        def ag_rdma(ch, h):
            s = h % SLOTS
            src = ch["own"] if h == 0 else ch["recv"].at[(h - 1) % SLOTS]
            return pltpu.make_async_remote_copy(
                src_ref=src, dst_ref=ch["recv"].at[s],
                send_sem=ch["ssem"].at[s], recv_sem=ch["rsem"].at[s],
                device_id=(ch["peer"],),
                device_id_type=pl.DeviceIdType.MESH,
            )

        import contextlib
        scope = jax.named_scope if _PROFILE_SCOPES else (
            lambda name: contextlib.nullcontext())
        with scope("gemm_own_and_barrier"):
            wb = w_ref[:, :].astype(jnp.bfloat16)
            wlo = wb[:, 0:half]
            whi = wb[:, half:n]
            out_ref[rows(my), :] = jnp.dot(
                x_ref[pl.ds(my * chunk, chunk), :].astype(jnp.bfloat16),
                wb, preferred_element_type=jnp.float32,
            )
            for ci in order:
                ch = rs[ci]
                ch["send"][0, :, :] = out_ref[rows(my), ch["cols"]].astype(
                    jnp.bfloat16)

            barrier_sem = pltpu.get_barrier_semaphore()
            for nbr in (left, right):
                ch_signal(barrier_sem, nbr)
            pl.semaphore_wait(barrier_sem, 2)

            if _DO_RS:
                for ci in order:
                    r = rs_rdma(rs[ci], 0)
                    r.start()
                    rs[ci]["rdma"] = [r]

        for h in range(n_hops) if _DO_RS else ():
            with scope(f"rs#hop={h}"):
                cf = (my - h - 1) % N_DEV
                cr = (my + h + 1) % N_DEV
                out_ref[rows(cf), pl.ds(0, half)] = jnp.dot(
                    x_ref[pl.ds(cf * chunk, chunk), :].astype(jnp.bfloat16),
                    wlo, preferred_element_type=jnp.float32,
                )
                out_ref[rows(cr), pl.ds(half, half)] = jnp.dot(
                    x_ref[pl.ds(cr * chunk, chunk), :].astype(jnp.bfloat16),
                    whi, preferred_element_type=jnp.float32,
                )
                for ci in order:
                    ch = rs[ci]
                    c = cf if ch["fwd"] else cr
                    ch["rdma"][h].wait()
                    acc = (ch["recv"][h % SLOTS, :, :].astype(jnp.float32)
                           + out_ref[rows(c), ch["cols"]])
                    if h < n_hops - 1:
                        ch["send"][(h + 1) % SLOTS, :, :] = acc.astype(
                            jnp.bfloat16)
                        if h + 1 >= SLOTS:
                            pl.semaphore_wait(ch["credit"], 1)
                        nr = rs_rdma(ch, h + 1)
                        nr.start()
                        ch["rdma"].append(nr)
                    else:
                        out_ref[rows(c), ch["cols"]] = acc
                    if h <= n_hops - 1 - SLOTS:
                        ch_signal(ch["credit"], ch["up"])

        if not _DO_RS:
            for h in range(n_hops):
                cf = (my - h - 1) % N_DEV
                cr = (my + h + 1) % N_DEV
                out_ref[rows(cf), pl.ds(0, half)] = jnp.dot(
                    x_ref[pl.ds(cf * chunk, chunk), :].astype(jnp.bfloat16),
                    wlo, preferred_element_type=jnp.float32,
                )
                out_ref[rows(cr), pl.ds(half, half)] = jnp.dot(
                    x_ref[pl.ds(cr * chunk, chunk), :].astype(jnp.bfloat16),
                    whi, preferred_element_type=jnp.float32,
                )

        own_f = (my + 1) % N_DEV
        own_r = (my + N_DEV - 1) % N_DEV

        with scope("bcast"):
            amax_own = jnp.maximum(
                jnp.max(jnp.abs(out_ref[rows(own_f), pl.ds(0, half)])),
                jnp.max(jnp.abs(out_ref[rows(own_r), pl.ds(half, half)])),
            )
            bcast[0, :, :] = jnp.full((8, 128), amax_own, jnp.float32)
            if _DO_BC:
                bsends = []
                for k in range(1, N_DEV):
                    s = N_DEV - k
                    b = pltpu.make_async_remote_copy(
                        src_ref=bcast.at[0], dst_ref=bcast.at[s],
                        send_sem=bsend_sems.at[s], recv_sem=brecv_sems.at[s],
                        device_id=((my + k) % N_DEV,),
                        device_id_type=pl.DeviceIdType.MESH,
                    )
                    b.start()
                    bsends.append(b)
                for s in range(1, N_DEV):
                    pltpu.make_async_remote_copy(
                        src_ref=bcast.at[0], dst_ref=bcast.at[s],
                        send_sem=bsend_sems.at[0], recv_sem=brecv_sems.at[s],
                        device_id=(left,),
                        device_id_type=pl.DeviceIdType.MESH,
                    ).wait_recv()
                for b in bsends:
                    b.wait_send()
                amax_g = jnp.max(bcast[:, :, :])
            else:
                amax_g = amax_own
            scale = amax_g / 127.0

        with scope("quant"):
            yf = out_ref[rows(own_f), pl.ds(0, half)]
            qf = jnp.clip(jnp.round(yf / scale), -127.0, 127.0)
            ag[0]["own"][:, :] = qf[:, 0:quar].astype(jnp.int8)
            ag[1]["own"][:, :] = qf[:, quar:half].astype(jnp.int8)
            out_ref[rows(own_f), pl.ds(0, half)] = qf * scale
            yr = out_ref[rows(own_r), pl.ds(half, half)]
            qr = jnp.clip(jnp.round(yr / scale), -127.0, 127.0)
            ag[2]["own"][:, :] = qr[:, 0:quar].astype(jnp.int8)
            ag[3]["own"][:, :] = qr[:, quar:half].astype(jnp.int8)
            out_ref[rows(own_r), pl.ds(half, half)] = qr * scale

        if _DO_AG:
            for ci in order:
                r = ag_rdma(ag[ci], 0)
                r.start()
                ag[ci]["rdma"] = [r]
        for h in range(n_hops) if _DO_AG else ():
            with scope(f"ag#hop={h}"):
                for ci in order:
                    ch = ag[ci]
                    c = (my - h) % N_DEV if ch["fwd"] else (my + h) % N_DEV
                    ch["rdma"][h].wait()
                    if 1 <= h and (h - 1) <= n_hops - 1 - SLOTS:
                        ch_signal(ch["credit"], ch["up"])
                    if h < n_hops - 1:
                        if h + 1 >= SLOTS:
                            pl.semaphore_wait(ch["credit"], 1)
                        nr = ag_rdma(ch, h + 1)
                        nr.start()
                        ch["rdma"].append(nr)
                    out_ref[rows(c), ch["cols"]] = (
                        ch["recv"][h % SLOTS, :, :].astype(jnp.float32)
                        * scale)

    rs_scratch = []
    for _ in range(N_CH):
        rs_scratch += [
            pltpu.VMEM((SLOTS, chunk, quar), jnp.bfloat16),
            pltpu.VMEM((SLOTS, chunk, quar), jnp.bfloat16),
            pltpu.SemaphoreType.DMA((SLOTS,)),
            pltpu.SemaphoreType.DMA((SLOTS,)),
            pltpu.SemaphoreType.REGULAR,
        ]
    ag_scratch = []
    for _ in range(N_CH):
        ag_scratch += [
            pltpu.VMEM((chunk, quar), jnp.int8),
            pltpu.VMEM((SLOTS, chunk, quar), jnp.int8),
            pltpu.SemaphoreType.DMA((SLOTS,)),
            pltpu.SemaphoreType.DMA((SLOTS,)),
            pltpu.SemaphoreType.REGULAR,
        ]
    return pl.pallas_call(
        body,
        out_shape=jax.ShapeDtypeStruct((m, n), jnp.float32),
        in_specs=[
            pl.BlockSpec(memory_space=pltpu.VMEM),
            pl.BlockSpec(memory_space=pltpu.VMEM),
        ],
        out_specs=pl.BlockSpec(memory_space=pltpu.VMEM),
        scratch_shapes=rs_scratch + [
            pltpu.VMEM((N_DEV, 8, 128), jnp.float32),
            pltpu.SemaphoreType.DMA((N_DEV,)),
            pltpu.SemaphoreType.DMA((N_DEV,)),
        ] + ag_scratch,
        compiler_params=pltpu.CompilerParams(
            collective_id=0,
            vmem_limit_bytes=60 * 1024 * 1024,
        ),
    )(x, w_mat)


# device time: 189566 ns/iter; 1.5918x vs baseline; 1.0116x over previous
import jax
import jax.numpy as jnp
from jax import lax
from jax.experimental import pallas as pl
from jax.experimental.pallas import tpu as pltpu

N_DEV = 16
N_CH = 4
N_AG_CH = 8
SLOTS = 4

import os
_PROFILE_SCOPES = bool(int(os.environ.get("KERNEL_PROFILE_SCOPES", "0")))
_ABLATE = set(os.environ.get("KERNEL_ABLATE", "").split(","))
_DO_RS = "rs" not in _ABLATE
_DO_AG = "ag" not in _ABLATE
_DO_BC = "bc" not in _ABLATE


def kernel(x, w_mat):
    m, _ = x.shape
    _, n = w_mat.shape
    chunk = m // N_DEV
    half = n // 2
    quar = n // 4
    n_hops = N_DEV - 1

    def body(x_ref, w_ref, out_ref, *scr):
        rs = [dict(send=scr[5 * i], recv=scr[5 * i + 1], ssem=scr[5 * i + 2],
                   rsem=scr[5 * i + 3], credit=scr[5 * i + 4])
              for i in range(N_CH)]
        bcast, bsend_sems, brecv_sems = scr[20], scr[21], scr[22]
        ag = [dict(own=scr[23 + 5 * i], recv=scr[24 + 5 * i],
                   ssem=scr[25 + 5 * i], rsem=scr[26 + 5 * i],
                   credit=scr[27 + 5 * i])
              for i in range(N_AG_CH)]

        my = lax.axis_index("i")
        left = (my + N_DEV - 1) % N_DEV
        right = (my + 1) % N_DEV

        def rows(c):
            return pl.ds(c * chunk, chunk)

        for ci in range(N_CH):
            fwd = ci < 2
            rs[ci]["cols"] = pl.ds(quar * ci, quar)
            rs[ci]["peer"] = right if fwd else left
            rs[ci]["up"] = left if fwd else right
            rs[ci]["fwd"] = fwd
        eighth = n // N_AG_CH
        for ci in range(N_AG_CH):
            fwd = ci < 4
            ag[ci]["cols"] = pl.ds(eighth * ci, eighth)
            ag[ci]["peer"] = right if fwd else left
            ag[ci]["up"] = left if fwd else right
            ag[ci]["fwd"] = fwd
        order = [0, 2, 1, 3]
        ag_order = [0, 4, 1, 5, 2, 6, 3, 7]

        def ch_signal(sem, dev):
            pl.semaphore_signal(
                sem, inc=1, device_id=(dev,),
                device_id_type=pl.DeviceIdType.MESH,
            )

        def rs_rdma(ch, h):
            s = h % SLOTS
            return pltpu.make_async_remote_copy(
                src_ref=ch["send"].at[s], dst_ref=ch["recv"].at[s],
                send_sem=ch["ssem"].at[s], recv_sem=ch["rsem"].at[s],
                device_id=(ch["peer"],),
                device_id_type=pl.DeviceIdType.MESH,
            )

        def ag_rdma(ch, h):
            s = h % SLOTS
            src = ch["own"] if h == 0 else ch["recv"].at[(h - 1) % SLOTS]
            return pltpu.make_async_remote_copy(
                src_ref=src, dst_ref=ch["recv"].at[s],
                send_sem=ch["ssem"].at[s], recv_sem=ch["rsem"].at[s],
                device_id=(ch["peer"],),
                device_id_type=pl.DeviceIdType.MESH,
            )

        import contextlib
        scope = jax.named_scope if _PROFILE_SCOPES else (
            lambda name: contextlib.nullcontext())
        with scope("gemm_own_and_barrier"):
            wb = w_ref[:, :].astype(jnp.bfloat16)
            wlo = wb[:, 0:half]
            whi = wb[:, half:n]
            out_ref[rows(my), :] = jnp.dot(
                x_ref[pl.ds(my * chunk, chunk), :].astype(jnp.bfloat16),
                wb, preferred_element_type=jnp.float32,
            )
            for ci in order:
                ch = rs[ci]
                ch["send"][0, :, :] = out_ref[rows(my), ch["cols"]].astype(
                    jnp.bfloat16)

            barrier_sem = pltpu.get_barrier_semaphore()
            for nbr in (left, right):
                ch_signal(barrier_sem, nbr)
            pl.semaphore_wait(barrier_sem, 2)

            if _DO_RS:
                for ci in order:
                    r = rs_rdma(rs[ci], 0)
                    r.start()
                    rs[ci]["rdma"] = [r]

        for h in range(n_hops) if _DO_RS else ():
            with scope(f"rs#hop={h}"):
                cf = (my - h - 1) % N_DEV
                cr = (my + h + 1) % N_DEV
                out_ref[rows(cf), pl.ds(0, half)] = jnp.dot(
                    x_ref[pl.ds(cf * chunk, chunk), :].astype(jnp.bfloat16),
                    wlo, preferred_element_type=jnp.float32,
                )
                out_ref[rows(cr), pl.ds(half, half)] = jnp.dot(
                    x_ref[pl.ds(cr * chunk, chunk), :].astype(jnp.bfloat16),
                    whi, preferred_element_type=jnp.float32,
                )
                for ci in order:
                    ch = rs[ci]
                    c = cf if ch["fwd"] else cr
                    ch["rdma"][h].wait()
                    acc = (ch["recv"][h % SLOTS, :, :].astype(jnp.float32)
                           + out_ref[rows(c), ch["cols"]])
                    if h < n_hops - 1:
                        ch["send"][(h + 1) % SLOTS, :, :] = acc.astype(
                            jnp.bfloat16)
                        if h + 1 >= SLOTS:
                            pl.semaphore_wait(ch["credit"], 1)
                        nr = rs_rdma(ch, h + 1)
                        nr.start()
                        ch["rdma"].append(nr)
                    else:
                        out_ref[rows(c), ch["cols"]] = acc
                    if h <= n_hops - 1 - SLOTS:
                        ch_signal(ch["credit"], ch["up"])

        if not _DO_RS:
            for h in range(n_hops):
                cf = (my - h - 1) % N_DEV
                cr = (my + h + 1) % N_DEV
                out_ref[rows(cf), pl.ds(0, half)] = jnp.dot(
                    x_ref[pl.ds(cf * chunk, chunk), :].astype(jnp.bfloat16),
                    wlo, preferred_element_type=jnp.float32,
                )
                out_ref[rows(cr), pl.ds(half, half)] = jnp.dot(
                    x_ref[pl.ds(cr * chunk, chunk), :].astype(jnp.bfloat16),
                    whi, preferred_element_type=jnp.float32,
                )

        own_f = (my + 1) % N_DEV
        own_r = (my + N_DEV - 1) % N_DEV

        with scope("bcast"):
            amax_own = jnp.maximum(
                jnp.max(jnp.abs(out_ref[rows(own_f), pl.ds(0, half)])),
                jnp.max(jnp.abs(out_ref[rows(own_r), pl.ds(half, half)])),
            )
            bcast[0, :, :] = jnp.full((8, 128), amax_own, jnp.float32)
            if _DO_BC:
                bsends = []
                for k in range(1, N_DEV):
                    s = N_DEV - k
                    b = pltpu.make_async_remote_copy(
                        src_ref=bcast.at[0], dst_ref=bcast.at[s],
                        send_sem=bsend_sems.at[s], recv_sem=brecv_sems.at[s],
                        device_id=((my + k) % N_DEV,),
                        device_id_type=pl.DeviceIdType.MESH,
                    )
                    b.start()
                    bsends.append(b)
                for s in range(1, N_DEV):
                    pltpu.make_async_remote_copy(
                        src_ref=bcast.at[0], dst_ref=bcast.at[s],
                        send_sem=bsend_sems.at[0], recv_sem=brecv_sems.at[s],
                        device_id=(left,),
                        device_id_type=pl.DeviceIdType.MESH,
                    ).wait_recv()
                for b in bsends:
                    b.wait_send()
                amax_g = jnp.max(bcast[:, :, :])
            else:
                amax_g = amax_own
            scale = amax_g / 127.0

        with scope("quant"):
            yf = out_ref[rows(own_f), pl.ds(0, half)]
            qf = jnp.clip(jnp.round(yf / scale), -127.0, 127.0)
            for i in range(4):
                ag[i]["own"][:, :] = qf[:, eighth * i:eighth * (i + 1)].astype(
                    jnp.int8)
            out_ref[rows(own_f), pl.ds(0, half)] = qf * scale
            yr = out_ref[rows(own_r), pl.ds(half, half)]
            qr = jnp.clip(jnp.round(yr / scale), -127.0, 127.0)
            for i in range(4):
                ag[4 + i]["own"][:, :] = qr[:, eighth * i:eighth * (i + 1)
                                            ].astype(jnp.int8)
            out_ref[rows(own_r), pl.ds(half, half)] = qr * scale

        if _DO_AG:
            for ci in ag_order:
                r = ag_rdma(ag[ci], 0)
                r.start()
                ag[ci]["rdma"] = [r]
        for h in range(n_hops) if _DO_AG else ():
            with scope(f"ag#hop={h}"):
                for ci in ag_order:
                    ch = ag[ci]
                    c = (my - h) % N_DEV if ch["fwd"] else (my + h) % N_DEV
                    ch["rdma"][h].wait()
                    if 1 <= h and (h - 1) <= n_hops - 1 - SLOTS:
                        ch_signal(ch["credit"], ch["up"])
                    if h < n_hops - 1:
                        if h + 1 >= SLOTS:
                            pl.semaphore_wait(ch["credit"], 1)
                        nr = ag_rdma(ch, h + 1)
                        nr.start()
                        ch["rdma"].append(nr)
                    out_ref[rows(c), ch["cols"]] = (
                        ch["recv"][h % SLOTS, :, :].astype(jnp.float32)
                        * scale)

    rs_scratch = []
    for _ in range(N_CH):
        rs_scratch += [
            pltpu.VMEM((SLOTS, chunk, quar), jnp.bfloat16),
            pltpu.VMEM((SLOTS, chunk, quar), jnp.bfloat16),
            pltpu.SemaphoreType.DMA((SLOTS,)),
            pltpu.SemaphoreType.DMA((SLOTS,)),
            pltpu.SemaphoreType.REGULAR,
        ]
    ag_scratch = []
    for _ in range(N_AG_CH):
        ag_scratch += [
            pltpu.VMEM((chunk, n // N_AG_CH), jnp.int8),
            pltpu.VMEM((SLOTS, chunk, n // N_AG_CH), jnp.int8),
            pltpu.SemaphoreType.DMA((SLOTS,)),
            pltpu.SemaphoreType.DMA((SLOTS,)),
            pltpu.SemaphoreType.REGULAR,
        ]
    return pl.pallas_call(
        body,
        out_shape=jax.ShapeDtypeStruct((m, n), jnp.float32),
        in_specs=[
            pl.BlockSpec(memory_space=pltpu.VMEM),
            pl.BlockSpec(memory_space=pltpu.VMEM),
        ],
        out_specs=pl.BlockSpec(memory_space=pltpu.VMEM),
        scratch_shapes=rs_scratch + [
            pltpu.VMEM((N_DEV, 8, 128), jnp.float32),
            pltpu.SemaphoreType.DMA((N_DEV,)),
            pltpu.SemaphoreType.DMA((N_DEV,)),
        ] + ag_scratch,
        compiler_params=pltpu.CompilerParams(
            collective_id=0,
            vmem_limit_bytes=60 * 1024 * 1024,
        ),
    )(x, w_mat)


# device time: 49889 ns/iter; 6.0486x vs baseline; 3.7998x over previous
import jax
import jax.numpy as jnp
from jax import lax
from jax.experimental import pallas as pl
from jax.experimental.pallas import tpu as pltpu

N_DEV = 16
N_CH = 8
N_AG_CH = 8
SLOTS = 4

import os
_PROFILE_SCOPES = bool(int(os.environ.get("KERNEL_PROFILE_SCOPES", "0")))
_ABLATE = set(os.environ.get("KERNEL_ABLATE", "").split(","))
_DO_RS = "rs" not in _ABLATE
_DO_AG = "ag" not in _ABLATE
_DO_BC = "bc" not in _ABLATE


def kernel(x, w_mat):
    m, _ = x.shape
    _, n = w_mat.shape
    chunk = m // N_DEV
    half = n // 2
    quar = n // 4
    n_hops = N_DEV - 1

    def body(x_ref, w_ref, out_ref, *scr):
        rs = [dict(send=scr[5 * i], recv=scr[5 * i + 1], ssem=scr[5 * i + 2],
                   rsem=scr[5 * i + 3], credit=scr[5 * i + 4])
              for i in range(N_CH)]
        b0 = 5 * N_CH
        bcast, bsend_sems, brecv_sems = scr[b0], scr[b0 + 1], scr[b0 + 2]
        a0 = b0 + 3
        ag = [dict(own=scr[a0 + 5 * i], recv=scr[a0 + 5 * i + 1],
                   ssem=scr[a0 + 5 * i + 2], rsem=scr[a0 + 5 * i + 3],
                   credit=scr[a0 + 5 * i + 4])
              for i in range(N_AG_CH)]

        my = lax.axis_index("i")
        left = (my + N_DEV - 1) % N_DEV
        right = (my + 1) % N_DEV

        def rows(c):
            return pl.ds(c * chunk, chunk)

        eighth = n // N_AG_CH
        for ci in range(N_CH):
            fwd = ci < N_CH // 2
            rs[ci]["cols"] = pl.ds(eighth * ci, eighth)
            rs[ci]["peer"] = right if fwd else left
            rs[ci]["up"] = left if fwd else right
            rs[ci]["fwd"] = fwd
        for ci in range(N_AG_CH):
            fwd = ci < N_AG_CH // 2
            ag[ci]["cols"] = pl.ds(eighth * ci, eighth)
            ag[ci]["peer"] = right if fwd else left
            ag[ci]["up"] = left if fwd else right
            ag[ci]["fwd"] = fwd
        order = [0, 4, 1, 5, 2, 6, 3, 7]
        ag_order = [0, 4, 1, 5, 2, 6, 3, 7]

        def ch_signal(sem, dev):
            pl.semaphore_signal(
                sem, inc=1, device_id=(dev,),
                device_id_type=pl.DeviceIdType.MESH,
            )

        def rs_rdma(ch, h):
            s = h % SLOTS
            return pltpu.make_async_remote_copy(
                src_ref=ch["send"].at[s], dst_ref=ch["recv"].at[s],
                send_sem=ch["ssem"].at[s], recv_sem=ch["rsem"].at[s],
                device_id=(ch["peer"],),
                device_id_type=pl.DeviceIdType.MESH,
            )

        def ag_rdma(ch, h):
            s = h % SLOTS
            src = ch["own"] if h == 0 else ch["recv"].at[(h - 1) % SLOTS]
            return pltpu.make_async_remote_copy(
                src_ref=src, dst_ref=ch["recv"].at[s],
                send_sem=ch["ssem"].at[s], recv_sem=ch["rsem"].at[s],
                device_id=(ch["peer"],),
                device_id_type=pl.DeviceIdType.MESH,
            )

        import contextlib
        scope = jax.named_scope if _PROFILE_SCOPES else (
            lambda name: contextlib.nullcontext())
        with scope("gemm_own_and_barrier"):
            wb = w_ref[:, :].astype(jnp.bfloat16)
            wlo = wb[:, 0:half]
            whi = wb[:, half:n]
            out_ref[rows(my), :] = jnp.dot(
                x_ref[pl.ds(my * chunk, chunk), :].astype(jnp.bfloat16),
                wb, preferred_element_type=jnp.float32,
            )
            for ci in order:
                ch = rs[ci]
                ch["send"][0, :, :] = out_ref[rows(my), ch["cols"]].astype(
                    jnp.bfloat16)

            barrier_sem = pltpu.get_barrier_semaphore()
            for nbr in (left, right):
                ch_signal(barrier_sem, nbr)
            pl.semaphore_wait(barrier_sem, 2)

            if _DO_RS:
                for ci in order:
                    r = rs_rdma(rs[ci], 0)
                    r.start()
                    rs[ci]["rdma"] = [r]

        for h in range(n_hops) if _DO_RS else ():
            with scope(f"rs#hop={h}"):
                cf = (my - h - 1) % N_DEV
                cr = (my + h + 1) % N_DEV
                out_ref[rows(cf), pl.ds(0, half)] = jnp.dot(
                    x_ref[pl.ds(cf * chunk, chunk), :].astype(jnp.bfloat16),
                    wlo, preferred_element_type=jnp.float32,
                )
                out_ref[rows(cr), pl.ds(half, half)] = jnp.dot(
                    x_ref[pl.ds(cr * chunk, chunk), :].astype(jnp.bfloat16),
                    whi, preferred_element_type=jnp.float32,
                )
                for ci in order:
                    ch = rs[ci]
                    c = cf if ch["fwd"] else cr
                    ch["rdma"][h].wait()
                    acc = (ch["recv"][h % SLOTS, :, :].astype(jnp.float32)
                           + out_ref[rows(c), ch["cols"]])
                    if h < n_hops - 1:
                        ch["send"][(h + 1) % SLOTS, :, :] = acc.astype(
                            jnp.bfloat16)
                        if h + 1 >= SLOTS:
                            pl.semaphore_wait(ch["credit"], 1)
                        nr = rs_rdma(ch, h + 1)
                        nr.start()
                        ch["rdma"].append(nr)
                    else:
                        out_ref[rows(c), ch["cols"]] = acc
                    if h <= n_hops - 1 - SLOTS:
                        ch_signal(ch["credit"], ch["up"])

        if not _DO_RS:
            for h in range(n_hops):
                cf = (my - h - 1) % N_DEV
                cr = (my + h + 1) % N_DEV
                out_ref[rows(cf), pl.ds(0, half)] = jnp.dot(
                    x_ref[pl.ds(cf * chunk, chunk), :].astype(jnp.bfloat16),
                    wlo, preferred_element_type=jnp.float32,
                )
                out_ref[rows(cr), pl.ds(half, half)] = jnp.dot(
                    x_ref[pl.ds(cr * chunk, chunk), :].astype(jnp.bfloat16),
                    whi, preferred_element_type=jnp.float32,
                )

        own_f = (my + 1) % N_DEV
        own_r = (my + N_DEV - 1) % N_DEV

        with scope("bcast"):
            amax_own = jnp.maximum(
                jnp.max(jnp.abs(out_ref[rows(own_f), pl.ds(0, half)])),
                jnp.max(jnp.abs(out_ref[rows(own_r), pl.ds(half, half)])),
            )
            bcast[0, :, :] = jnp.full((8, 128), amax_own, jnp.float32)
            if _DO_BC:
                bsends = []
                for k in range(1, N_DEV):
                    s = N_DEV - k
                    b = pltpu.make_async_remote_copy(
                        src_ref=bcast.at[0], dst_ref=bcast.at[s],
                        send_sem=bsend_sems.at[s], recv_sem=brecv_sems.at[s],
                        device_id=((my + k) % N_DEV,),
                        device_id_type=pl.DeviceIdType.MESH,
                    )
                    b.start()
                    bsends.append(b)
                for s in range(1, N_DEV):
                    pltpu.make_async_remote_copy(
                        src_ref=bcast.at[0], dst_ref=bcast.at[s],
                        send_sem=bsend_sems.at[0], recv_sem=brecv_sems.at[s],
                        device_id=(left,),
                        device_id_type=pl.DeviceIdType.MESH,
                    ).wait_recv()
                for b in bsends:
                    b.wait_send()
                amax_g = jnp.max(bcast[:, :, :])
            else:
                amax_g = amax_own
            scale = amax_g / 127.0

        with scope("quant"):
            yf = out_ref[rows(own_f), pl.ds(0, half)]
            qf = jnp.clip(jnp.round(yf / scale), -127.0, 127.0)
            for i in range(4):
                ag[i]["own"][:, :] = qf[:, eighth * i:eighth * (i + 1)].astype(
                    jnp.int8)
            out_ref[rows(own_f), pl.ds(0, half)] = qf * scale
            yr = out_ref[rows(own_r), pl.ds(half, half)]
            qr = jnp.clip(jnp.round(yr / scale), -127.0, 127.0)
            for i in range(4):
                ag[4 + i]["own"][:, :] = qr[:, eighth * i:eighth * (i + 1)
                                            ].astype(jnp.int8)
            out_ref[rows(own_r), pl.ds(half, half)] = qr * scale

        if _DO_AG:
            for ci in ag_order:
                r = ag_rdma(ag[ci], 0)
                r.start()
                ag[ci]["rdma"] = [r]
        for h in range(n_hops) if _DO_AG else ():
            with scope(f"ag#hop={h}"):
                for ci in ag_order:
                    ch = ag[ci]
                    c = (my - h) % N_DEV if ch["fwd"] else (my + h) % N_DEV
                    ch["rdma"][h].wait()
                    if 1 <= h and (h - 1) <= n_hops - 1 - SLOTS:
                        ch_signal(ch["credit"], ch["up"])
                    if h < n_hops - 1:
                        if h + 1 >= SLOTS:
                            pl.semaphore_wait(ch["credit"], 1)
                        nr = ag_rdma(ch, h + 1)
                        nr.start()
                        ch["rdma"].append(nr)
                    out_ref[rows(c), ch["cols"]] = (
                        ch["recv"][h % SLOTS, :, :].astype(jnp.float32)
                        * scale)

    rs_scratch = []
    for _ in range(N_CH):
        rs_scratch += [
            pltpu.VMEM((SLOTS, chunk, n // N_CH), jnp.bfloat16),
            pltpu.VMEM((SLOTS, chunk, n // N_CH), jnp.bfloat16),
            pltpu.SemaphoreType.DMA((SLOTS,)),
            pltpu.SemaphoreType.DMA((SLOTS,)),
            pltpu.SemaphoreType.REGULAR,
        ]
    ag_scratch = []
    for _ in range(N_AG_CH):
        ag_scratch += [
            pltpu.VMEM((chunk, n // N_AG_CH), jnp.int8),
            pltpu.VMEM((SLOTS, chunk, n // N_AG_CH), jnp.int8),
            pltpu.SemaphoreType.DMA((SLOTS,)),
            pltpu.SemaphoreType.DMA((SLOTS,)),
            pltpu.SemaphoreType.REGULAR,
        ]
    return pl.pallas_call(
        body,
        out_shape=jax.ShapeDtypeStruct((m, n), jnp.float32),
        in_specs=[
            pl.BlockSpec(memory_space=pltpu.VMEM),
            pl.BlockSpec(memory_space=pltpu.VMEM),
        ],
        out_specs=pl.BlockSpec(memory_space=pltpu.VMEM),
        scratch_shapes=rs_scratch + [
            pltpu.VMEM((N_DEV, 8, 128), jnp.float32),
            pltpu.SemaphoreType.DMA((N_DEV,)),
            pltpu.SemaphoreType.DMA((N_DEV,)),
        ] + ag_scratch,
        compiler_params=pltpu.CompilerParams(
            collective_id=0,
            vmem_limit_bytes=60 * 1024 * 1024,
        ),
    )(x, w_mat)
